# baseline (speedup 1.0000x reference)
import jax
import jax.numpy as jnp
from jax.experimental import pallas as pl
from jax.experimental.pallas import tpu as pltpu

ALPHA = 0.2
EPS = 1e-16
LOG2E = 1.4426950408889634


def _gat_block_kernel(h_ref, adj_ref, w_ref, a_ref, out_ref,
                      wh_ref, s1l_ref, s1al_ref, s2t_ref):
    dout = w_ref.shape[1]
    blk = adj_ref.shape[1]

    @pl.when(pl.program_id(0) == 0)
    def _():
        wh0 = jnp.dot(h_ref[...], w_ref[...], preferred_element_type=jnp.float32)
        wh_ref[...] = jnp.concatenate(
            [wh0, jnp.ones_like(wh0[:, :1]), jnp.zeros_like(wh0[:, 1:])],
            axis=1).astype(jnp.bfloat16)
        s1 = jnp.dot(wh0, a_ref[:dout, :],
                     preferred_element_type=jnp.float32) * LOG2E
        s1l_ref[...] = s1
        s1al_ref[...] = ALPHA * s1
        s2t_ref[...] = (jnp.dot(wh0, a_ref[dout:, :],
                                preferred_element_type=jnp.float32) * LOG2E).T

    j0 = pl.program_id(0) * blk
    n = adj_ref.shape[0]
    half = n // 2
    s2b = s2t_ref[:, pl.ds(j0, blk)]
    mask_t = adj_ref[0:half, :] > 0.0
    mask_b = adj_ref[half:n, :] > 0.0
    s1l_t = s1l_ref[0:half, :]
    s1l_b = s1l_ref[half:n, :]
    s1al_t = s1al_ref[0:half, :]
    s1al_b = s1al_ref[half:n, :]
    m1 = jnp.maximum(
        jnp.max(jnp.where(mask_t, s1l_t, -jnp.inf), axis=0, keepdims=True),
        jnp.max(jnp.where(mask_b, s1l_b, -jnp.inf), axis=0, keepdims=True))
    mb = m1 + s2b
    m = jnp.maximum(mb, ALPHA * mb)
    c1 = s2b - m
    c2 = ALPHA * s2b - m
    dims = (((0,), (0,)), ((), ()))
    t_t = jnp.maximum(s1l_t + c1, s1al_t + c2)
    p_t = jnp.exp2(jnp.where(mask_t, t_t, -jnp.inf))
    hp_t = jax.lax.dot_general(
        p_t.astype(jnp.bfloat16), wh_ref[0:half, :], dims,
        preferred_element_type=jnp.float32)
    t_b = jnp.maximum(s1l_b + c1, s1al_b + c2)
    p_b = jnp.exp2(jnp.where(mask_b, t_b, -jnp.inf))
    hp_aug = hp_t + jax.lax.dot_general(
        p_b.astype(jnp.bfloat16), wh_ref[half:n, :], dims,
        preferred_element_type=jnp.float32)
    denom = hp_aug[:, dout:dout + 1] + EPS
    hp = hp_aug[:, :dout] * (1.0 / denom)
    out_ref[...] = jnp.where(hp > 0.0, hp, jnp.exp(hp) - 1.0)


def kernel(h, adj, W, a):
    N, din = h.shape
    dout = W.shape[1]
    blk = 512
    grid = N // blk
    return pl.pallas_call(
        _gat_block_kernel,
        grid=(grid,),
        in_specs=[
            pl.BlockSpec((N, din), lambda i: (0, 0)),
            pl.BlockSpec((N, blk), lambda i: (0, i)),
            pl.BlockSpec((din, dout), lambda i: (0, 0)),
            pl.BlockSpec((2 * dout, 1), lambda i: (0, 0)),
        ],
        out_specs=pl.BlockSpec((blk, dout), lambda i: (i, 0)),
        out_shape=jax.ShapeDtypeStruct((N, dout), jnp.float32),
        scratch_shapes=[
            pltpu.VMEM((N, 2 * dout), jnp.bfloat16),
            pltpu.VMEM((N, 1), jnp.float32),
            pltpu.VMEM((N, 1), jnp.float32),
            pltpu.VMEM((1, N), jnp.float32),
        ],
    )(h, adj, W, a)


# speedup vs baseline: 1.0116x; 1.0116x over previous
import jax
import jax.numpy as jnp
from jax.experimental import pallas as pl
from jax.experimental.pallas import tpu as pltpu

ALPHA = 0.2
EPS = 1e-16
LOG2E = 1.4426950408889634


def _gat_block_kernel(h_ref, adj_ref, w_ref, a_ref, out_ref,
                      wh_ref, s1l_ref, s1al_ref, s2t_ref):
    dout = w_ref.shape[1]
    blk = adj_ref.shape[1]

    @pl.when(pl.program_id(0) == 0)
    def _():
        wh0 = jnp.dot(h_ref[...], w_ref[...], preferred_element_type=jnp.float32)
        wh_ref[...] = jnp.concatenate(
            [wh0, jnp.ones_like(wh0[:, :1]), jnp.zeros_like(wh0[:, 1:])],
            axis=1).astype(jnp.bfloat16)
        s1 = jnp.dot(wh0, a_ref[:dout, :],
                     preferred_element_type=jnp.float32) * LOG2E
        s1l_ref[...] = s1
        s1al_ref[...] = ALPHA * s1
        s2t_ref[...] = (jnp.dot(wh0, a_ref[dout:, :],
                                preferred_element_type=jnp.float32) * LOG2E).T

    j0 = pl.program_id(0) * blk
    s1l = s1l_ref[...]
    s1al = s1al_ref[...]
    s2b = s2t_ref[:, pl.ds(j0, blk)]
    mask = adj_ref[...] > 0.0
    m1 = jnp.max(jnp.where(mask, s1l, -jnp.inf), axis=0, keepdims=True)
    mb = m1 + s2b
    m = jnp.maximum(mb, ALPHA * mb)
    c1 = s2b - m
    c2 = ALPHA * s2b - m
    t = jnp.maximum(s1l + c1, s1al + c2)
    p = jnp.exp2(jnp.where(mask, t, -jnp.inf))
    hp_aug = jax.lax.dot_general(
        p.astype(jnp.bfloat16), wh_ref[...],
        (((0,), (0,)), ((), ())),
        preferred_element_type=jnp.float32,
    )
    denom = hp_aug[:, dout:dout + 1] + EPS
    hp = hp_aug[:, :dout] * (1.0 / denom)
    out_ref[...] = jnp.where(hp > 0.0, hp, jnp.exp(hp) - 1.0)


def kernel(h, adj, W, a):
    N, din = h.shape
    dout = W.shape[1]
    blk = 512
    grid = N // blk
    return pl.pallas_call(
        _gat_block_kernel,
        grid=(grid,),
        in_specs=[
            pl.BlockSpec((N, din), lambda i: (0, 0)),
            pl.BlockSpec((N, blk), lambda i: (0, i)),
            pl.BlockSpec((din, dout), lambda i: (0, 0)),
            pl.BlockSpec((2 * dout, 1), lambda i: (0, 0)),
        ],
        out_specs=pl.BlockSpec((blk, dout), lambda i: (i, 0)),
        out_shape=jax.ShapeDtypeStruct((N, dout), jnp.float32),
        scratch_shapes=[
            pltpu.VMEM((N, 2 * dout), jnp.bfloat16),
            pltpu.VMEM((N, 1), jnp.float32),
            pltpu.VMEM((N, 1), jnp.float32),
            pltpu.VMEM((1, N), jnp.float32),
        ],
    )(h, adj, W, a)
